# baseline (speedup 1.0000x reference)
import jax
import jax.numpy as jnp
from jax import lax
from jax.experimental import pallas as pl
from jax.experimental.pallas import tpu as pltpu
from jax.experimental.pallas import tpu_sc as plsc

NC = 2
NS = 16
LANES = 16
NW = NC * NS

B, L = 4096, 200
V, D = 1000000, 64
T = 2
EPS = 1e-12

N = B * L
RPW = N // NW
CHUNK = 256
NCH = RPW // CHUNK
SUB = 128
NSUB = CHUNK // SUB
GROUPS = CHUNK // LANES
NL = 8
OUTR = L * (D // 8)
OUTC = (B // 128) * 8 * 128


def _emb_body(ids, tts, ctab, word, gamma, beta, out,
              idx_v, tvec_v, cbig, gb_v, xbufs, obuf, gam_s, bet_s, gsems):
    wid = lax.axis_index("s") * NC + lax.axis_index("c")
    base = wid * RPW
    l0 = base // B

    pltpu.sync_copy(ids.at[pl.ds(base, RPW)], idx_v)
    pltpu.sync_copy(tts.at[pl.ds(base, RPW)], tvec_v)
    pltpu.sync_copy(ctab.at[0, pl.ds(l0, NL)], cbig.at[pl.ds(0, NL)])
    pltpu.sync_copy(ctab.at[1, pl.ds(l0, NL)], cbig.at[pl.ds(NL, NL)])
    pltpu.sync_copy(gamma, gb_v.at[0])
    pltpu.sync_copy(beta, gb_v.at[1])
    for j in range(D // LANES):
        gv = gb_v[0, pl.ds(j * LANES, LANES)]
        bv = gb_v[1, pl.ds(j * LANES, LANES)]
        for k in range(LANES):
            gam_s[j * LANES + k] = gv[k]
            bet_s[j * LANES + k] = bv[k]

    iota = lax.iota(jnp.int32, LANES)

    def issue_gather(chunk, xb, sem):
        for j in range(NSUB):
            pltpu.async_copy(
                word.at[idx_v.at[pl.ds(chunk * CHUNK + j * SUB, SUB)]],
                xb.at[pl.ds(j * SUB, SUB)], sem)

    def drain_gather(xb, sem):
        pltpu.make_async_copy(word.at[pl.ds(0, CHUNK)], xb, sem).wait()

    def compute(chunk, xb):
        gbase = base + chunk * CHUNK
        li = gbase // B - l0

        for jj in range(CHUNK // 128):
            @pl.loop(0, 128 // LANES)
            def _group(gi):
                g = jj * (128 // LANES) + gi
                r0 = g * LANES
                riota = iota + r0
                tvec = tvec_v[pl.ds(chunk * CHUNK + r0, LANES)]
                crow = tvec * NL + li
                c0 = gi * LANES

                s1 = [jnp.zeros((LANES,), jnp.float32) for _ in range(4)]
                s2 = [jnp.zeros((LANES,), jnp.float32) for _ in range(4)]
                for d in range(D):
                    dfull = jnp.full((LANES,), d, jnp.int32)
                    w_d = plsc.load_gather(xb, [riota, dfull])
                    c_d = plsc.load_gather(cbig, [crow, dfull])
                    x_d = w_d + c_d
                    obuf[d // 8, jj, d % 8, pl.ds(c0, LANES)] = x_d
                    s1[d % 4] = s1[d % 4] + x_d
                    s2[d % 4] = s2[d % 4] + x_d * x_d

                mean = ((s1[0] + s1[1]) + (s1[2] + s1[3])) * (1.0 / D)
                ex2 = ((s2[0] + s2[1]) + (s2[2] + s2[3])) * (1.0 / D)
                var = jnp.maximum(ex2 - mean * mean, 0.0) + EPS
                yi = (jnp.int32(0x5F3759DF)
                      - (plsc.bitcast(var, jnp.int32) >> 1))
                y = plsc.bitcast(yi, jnp.float32)
                for _ in range(3):
                    y = y * (1.5 - 0.5 * var * y * y)

                for d in range(D):
                    xj = obuf[d // 8, jj, d % 8, pl.ds(c0, LANES)]
                    obuf[d // 8, jj, d % 8, pl.ds(c0, LANES)] = (
                        (xj - mean) * y * gam_s[d] + bet_s[d])

    issue_gather(0, xbufs[0], gsems[0])

    @pl.loop(0, NCH, step=2)
    def _chunks(ci):
        for b in range(2):
            chunk = ci + b
            xb, sem = xbufs[b], gsems[b]
            if b == 0:
                issue_gather(chunk + 1, xbufs[1], gsems[1])
            else:
                @pl.when(chunk + 1 < NCH)
                def _():
                    issue_gather(chunk + 1, xbufs[0], gsems[0])
            drain_gather(xb, sem)
            compute(chunk, xb)
            gbase = base + chunk * CHUNK
            lc = gbase // B
            j0 = (gbase % B) // 128
            pltpu.sync_copy(obuf,
                            out.at[lc, :, pl.ds(j0, CHUNK // 128)])


@jax.jit
def _emb(ids, tts, ctab, word, gamma, beta):
    mesh = plsc.VectorSubcoreMesh(core_axis_name="c", subcore_axis_name="s",
                                  num_cores=NC, num_subcores=NS)
    return pl.kernel(
        _emb_body,
        out_type=jax.ShapeDtypeStruct((L, D // 8, B // 128, 8, 128),
                                      jnp.float32),
        mesh=mesh,
        compiler_params=pltpu.CompilerParams(needs_layout_passes=False,
                                             use_tc_tiling_on_sc=False),
        scratch_types=[
            pltpu.VMEM((RPW,), jnp.int32),
            pltpu.VMEM((RPW,), jnp.int32),
            pltpu.VMEM((2 * NL, D), jnp.float32),
            pltpu.VMEM((2, D), jnp.float32),
            [pltpu.VMEM((CHUNK, D), jnp.float32),
             pltpu.VMEM((CHUNK, D), jnp.float32)],
            pltpu.VMEM((D // 8, CHUNK // 128, 8, 128),
                       jnp.float32),
            pltpu.SMEM((D,), jnp.float32),
            pltpu.SMEM((D,), jnp.float32),
            [pltpu.SemaphoreType.DMA,
             pltpu.SemaphoreType.DMA],
        ],
    )(ids, tts, ctab, word, gamma, beta)


def kernel(input_ids, token_type_ids, word_table, pos_table, type_table,
           gamma, beta):
    ids = input_ids.astype(jnp.int32).T.reshape(N)
    tts = token_type_ids.astype(jnp.int32).T.reshape(N)
    ctab = jnp.zeros((T, L + NL, D), jnp.float32)
    ctab = ctab.at[:, :L, :].set(type_table[:, None, :]
                                 + pos_table[None, :L, :])
    q = _emb(ids, tts, ctab, word_table, gamma, beta)
    return q.transpose(2, 4, 0, 1, 3).reshape(B, L, D)


# speedup vs baseline: 1.6148x; 1.6148x over previous
import jax
import jax.numpy as jnp
from jax import lax
from jax.experimental import pallas as pl
from jax.experimental.pallas import tpu as pltpu
from jax.experimental.pallas import tpu_sc as plsc

NC = 2
NS = 16
LANES = 16
NW = NC * NS

B, L = 4096, 200
V, D = 1000000, 64
T = 2
EPS = 1e-12

N = B * L
RPW = N // NW
CHUNK = 256
NCH = RPW // CHUNK
SUB = 128
NSUB = CHUNK // SUB
GROUPS = CHUNK // LANES
DJ = D // LANES
NL = 8


def _emb_body(ids, tts, ctab, word, gamma, beta, out,
              idx_v, tvec_v, cbig, gb_v, xbufs, gsems):
    wid = lax.axis_index("s") * NC + lax.axis_index("c")
    base = wid * RPW
    l0 = base // B

    pltpu.sync_copy(ids.at[pl.ds(base, RPW)], idx_v)
    pltpu.sync_copy(tts.at[pl.ds(base, RPW)], tvec_v)
    pltpu.sync_copy(ctab.at[0, pl.ds(l0, NL)], cbig.at[pl.ds(0, NL)])
    pltpu.sync_copy(ctab.at[1, pl.ds(l0, NL)], cbig.at[pl.ds(NL, NL)])
    pltpu.sync_copy(gamma, gb_v.at[0])
    pltpu.sync_copy(beta, gb_v.at[1])

    gvecs = [gb_v[0, pl.ds(j * LANES, LANES)] for j in range(DJ)]
    bvecs = [gb_v[1, pl.ds(j * LANES, LANES)] for j in range(DJ)]
    lane15 = jnp.full((LANES,), LANES - 1, jnp.int32)
    kfulls = [jnp.full((LANES,), k, jnp.int32) for k in range(LANES)]

    def issue_gather(chunk, xb, sem):
        for j in range(NSUB):
            pltpu.async_copy(
                word.at[idx_v.at[pl.ds(chunk * CHUNK + j * SUB, SUB)]],
                xb.at[pl.ds(j * SUB, SUB)], sem)

    def drain_gather(xb, sem):
        pltpu.make_async_copy(word.at[pl.ds(0, CHUNK)], xb, sem).wait()

    def splat(vec, kfull):
        return vec.at[kfull].get(mode="promise_in_bounds")

    def compute(chunk, xb):
        li = (base + chunk * CHUNK) // B - l0
        c0s = [cbig[li, pl.ds(j * LANES, LANES)] for j in range(DJ)]
        dds = [cbig[NL + li, pl.ds(j * LANES, LANES)] - c0s[j]
               for j in range(DJ)]

        @pl.loop(0, GROUPS)
        def _group(g):
            r0 = g * LANES
            tvec = tvec_v[pl.ds(chunk * CHUNK + r0, LANES)]
            tf = tvec.astype(jnp.float32)
            for k in range(LANES):
                r = r0 + k
                tk = splat(tf, kfulls[k])
                xs = [xb[r, pl.ds(j * LANES, LANES)] + (c0s[j] + tk * dds[j])
                      for j in range(DJ)]
                tot = (xs[0] + xs[1]) + (xs[2] + xs[3])
                sq = [x * x for x in xs]
                tot2 = (sq[0] + sq[1]) + (sq[2] + sq[3])
                meanv = splat(plsc.cumsum(tot), lane15) * (1.0 / D)
                ex2v = splat(plsc.cumsum(tot2), lane15) * (1.0 / D)
                var = jnp.maximum(ex2v - meanv * meanv, 0.0) + EPS
                yi = (jnp.int32(0x5F3759DF)
                      - (plsc.bitcast(var, jnp.int32) >> 1))
                y = plsc.bitcast(yi, jnp.float32)
                for _ in range(2):
                    y = y * (1.5 - 0.5 * var * y * y)
                for j in range(DJ):
                    xb[r, pl.ds(j * LANES, LANES)] = (
                        (xs[j] - meanv) * y * gvecs[j] + bvecs[j])

    issue_gather(0, xbufs[0], gsems[0])

    @pl.loop(0, NCH, step=2)
    def _chunks(ci):
        for b in range(2):
            chunk = ci + b
            xb, sem = xbufs[b], gsems[b]
            if b == 0:
                issue_gather(chunk + 1, xbufs[1], gsems[1])
            else:
                @pl.when(chunk + 1 < NCH)
                def _():
                    issue_gather(chunk + 1, xbufs[0], gsems[0])
            drain_gather(xb, sem)
            compute(chunk, xb)
            gbase = base + chunk * CHUNK
            pltpu.sync_copy(xb, out.at[gbase // B,
                                       pl.ds(gbase % B, CHUNK)])


@jax.jit
def _emb(ids, tts, ctab, word, gamma, beta):
    mesh = plsc.VectorSubcoreMesh(core_axis_name="c", subcore_axis_name="s",
                                  num_cores=NC, num_subcores=NS)
    return pl.kernel(
        _emb_body,
        out_type=jax.ShapeDtypeStruct((L, B, D), jnp.float32),
        mesh=mesh,
        compiler_params=pltpu.CompilerParams(needs_layout_passes=False,
                                             use_tc_tiling_on_sc=False),
        scratch_types=[
            pltpu.VMEM((RPW,), jnp.int32),
            pltpu.VMEM((RPW,), jnp.int32),
            pltpu.VMEM((2 * NL, D), jnp.float32),
            pltpu.VMEM((2, D), jnp.float32),
            [pltpu.VMEM((CHUNK, D), jnp.float32),
             pltpu.VMEM((CHUNK, D), jnp.float32)],
            [pltpu.SemaphoreType.DMA,
             pltpu.SemaphoreType.DMA],
        ],
    )(ids, tts, ctab, word, gamma, beta)


def kernel(input_ids, token_type_ids, word_table, pos_table, type_table,
           gamma, beta):
    ids = input_ids.astype(jnp.int32).T.reshape(N)
    tts = token_type_ids.astype(jnp.int32).T.reshape(N)
    ctab = jnp.zeros((T, L + NL, D), jnp.float32)
    ctab = ctab.at[:, :L, :].set(type_table[:, None, :]
                                 + pos_table[None, :L, :])
    q = _emb(ids, tts, ctab, word_table, gamma, beta)
    return q.transpose(1, 0, 2)


# speedup vs baseline: 1.8811x; 1.1649x over previous
import jax
import jax.numpy as jnp
from jax import lax
from jax.experimental import pallas as pl
from jax.experimental.pallas import tpu as pltpu
from jax.experimental.pallas import tpu_sc as plsc

NC = 2
NS = 16
LANES = 16
NW = NC * NS

B, L = 4096, 200
V, D = 1000000, 64
T = 2
EPS = 1e-12

N = B * L
RPW = N // NW
CHUNK = 256
NCH = RPW // CHUNK
SUB = 128
NSUB = CHUNK // SUB
GROUPS = CHUNK // LANES
DJ = D // LANES
NL = 8


def _emb_body(ids, tts, ctab, word, gamma, beta, out,
              idx_v, tvec_v, cbig, gb_v, xbufs, srow_v, sqrow_v, gsems):
    wid = lax.axis_index("s") * NC + lax.axis_index("c")
    base = wid * RPW
    l0 = base // B

    pltpu.sync_copy(ids.at[pl.ds(base, RPW)], idx_v)
    pltpu.sync_copy(tts.at[pl.ds(base, RPW)], tvec_v)
    pltpu.sync_copy(ctab.at[0, pl.ds(l0, NL)], cbig.at[pl.ds(0, NL)])
    pltpu.sync_copy(ctab.at[1, pl.ds(l0, NL)], cbig.at[pl.ds(NL, NL)])
    pltpu.sync_copy(gamma, gb_v.at[0])
    pltpu.sync_copy(beta, gb_v.at[1])

    gvecs = [gb_v[0, pl.ds(j * LANES, LANES)] for j in range(DJ)]
    bvecs = [gb_v[1, pl.ds(j * LANES, LANES)] for j in range(DJ)]
    iota = lax.iota(jnp.int32, LANES)
    m15 = iota == (LANES - 1)
    kfulls = [jnp.full((LANES,), k, jnp.int32) for k in range(LANES)]

    def issue_gather(chunk, xb, sem):
        for j in range(NSUB):
            pltpu.async_copy(
                word.at[idx_v.at[pl.ds(chunk * CHUNK + j * SUB, SUB)]],
                xb.at[pl.ds(j * SUB, SUB)], sem)

    def drain_gather(xb, sem):
        pltpu.make_async_copy(word.at[pl.ds(0, CHUNK)], xb, sem).wait()

    def splat(vec, kfull):
        return vec.at[kfull].get(mode="promise_in_bounds")

    def compute(chunk, xb):
        li = (base + chunk * CHUNK) // B - l0
        c0s = [cbig[li, pl.ds(j * LANES, LANES)] for j in range(DJ)]
        c1s = [cbig[NL + li, pl.ds(j * LANES, LANES)] for j in range(DJ)]

        @pl.loop(0, GROUPS)
        def _group(g):
            r0 = g * LANES
            tvec = tvec_v[pl.ds(chunk * CHUNK + r0, LANES)]
            for k in range(LANES):
                r = r0 + k
                tmask = splat(tvec, kfulls[k]) != 0
                xs = [xb[r, pl.ds(j * LANES, LANES)]
                      + jnp.where(tmask, c1s[j], c0s[j])
                      for j in range(DJ)]
                tot = (xs[0] + xs[1]) + (xs[2] + xs[3])
                sq = [x * x for x in xs]
                tot2 = (sq[0] + sq[1]) + (sq[2] + sq[3])
                plsc.store_scatter(srow_v, [kfulls[k]], plsc.cumsum(tot),
                                   mask=m15)
                plsc.store_scatter(sqrow_v, [kfulls[k]], plsc.cumsum(tot2),
                                   mask=m15)
                for j in range(DJ):
                    xb[r, pl.ds(j * LANES, LANES)] = xs[j]
            mean16 = srow_v[...] * (1.0 / D)
            ex216 = sqrow_v[...] * (1.0 / D)
            var = jnp.maximum(ex216 - mean16 * mean16, 0.0) + EPS
            yi = (jnp.int32(0x5F3759DF)
                  - (plsc.bitcast(var, jnp.int32) >> 1))
            y = plsc.bitcast(yi, jnp.float32)
            for _ in range(3):
                y = y * (1.5 - 0.5 * var * y * y)
            for k in range(LANES):
                r = r0 + k
                meank = splat(mean16, kfulls[k])
                invk = splat(y, kfulls[k])
                for j in range(DJ):
                    xj = xb[r, pl.ds(j * LANES, LANES)]
                    xb[r, pl.ds(j * LANES, LANES)] = (
                        (xj - meank) * invk * gvecs[j] + bvecs[j])

    issue_gather(0, xbufs[0], gsems[0])

    @pl.loop(0, NCH, step=2)
    def _chunks(ci):
        for b in range(2):
            chunk = ci + b
            xb, sem = xbufs[b], gsems[b]
            if b == 0:
                issue_gather(chunk + 1, xbufs[1], gsems[1])
            else:
                @pl.when(chunk + 1 < NCH)
                def _():
                    issue_gather(chunk + 1, xbufs[0], gsems[0])
            drain_gather(xb, sem)
            compute(chunk, xb)
            gbase = base + chunk * CHUNK
            pltpu.sync_copy(xb, out.at[gbase // B,
                                       pl.ds(gbase % B, CHUNK)])


@jax.jit
def _emb(ids, tts, ctab, word, gamma, beta):
    mesh = plsc.VectorSubcoreMesh(core_axis_name="c", subcore_axis_name="s",
                                  num_cores=NC, num_subcores=NS)
    return pl.kernel(
        _emb_body,
        out_type=jax.ShapeDtypeStruct((L, B, D), jnp.float32),
        mesh=mesh,
        compiler_params=pltpu.CompilerParams(needs_layout_passes=False,
                                             use_tc_tiling_on_sc=False),
        scratch_types=[
            pltpu.VMEM((RPW,), jnp.int32),
            pltpu.VMEM((RPW,), jnp.int32),
            pltpu.VMEM((2 * NL, D), jnp.float32),
            pltpu.VMEM((2, D), jnp.float32),
            [pltpu.VMEM((CHUNK, D), jnp.float32),
             pltpu.VMEM((CHUNK, D), jnp.float32)],
            pltpu.VMEM((LANES,), jnp.float32),
            pltpu.VMEM((LANES,), jnp.float32),
            [pltpu.SemaphoreType.DMA,
             pltpu.SemaphoreType.DMA],
        ],
    )(ids, tts, ctab, word, gamma, beta)


def kernel(input_ids, token_type_ids, word_table, pos_table, type_table,
           gamma, beta):
    ids = input_ids.astype(jnp.int32).T.reshape(N)
    tts = token_type_ids.astype(jnp.int32).T.reshape(N)
    ctab = jnp.zeros((T, L + NL, D), jnp.float32)
    ctab = ctab.at[:, :L, :].set(type_table[:, None, :]
                                 + pos_table[None, :L, :])
    q = _emb(ids, tts, ctab, word_table, gamma, beta)
    return q.transpose(1, 0, 2)


# speedup vs baseline: 1.9445x; 1.0337x over previous
import jax
import jax.numpy as jnp
from jax import lax
from jax.experimental import pallas as pl
from jax.experimental.pallas import tpu as pltpu
from jax.experimental.pallas import tpu_sc as plsc

NC = 2
NS = 16
LANES = 16
NW = NC * NS

B, L = 4096, 200
V, D = 1000000, 64
T = 2
EPS = 1e-12

N = B * L
RPW = N // NW
CHUNK = 256
NCH = RPW // CHUNK
SUB = 128
NSUB = CHUNK // SUB
GROUPS = CHUNK // LANES
DJ = D // LANES
NL = 8


def _emb_body(ids, tts, ctab, word, gamma, beta, out,
              idx_v, tvec_v, cbig, gb_v, xbufs, srow_v, sqrow_v,
              gsems, osems):
    wid = lax.axis_index("s") * NC + lax.axis_index("c")
    base = wid * RPW
    l0 = base // B

    pltpu.sync_copy(ids.at[pl.ds(base, RPW)], idx_v)
    pltpu.sync_copy(tts.at[pl.ds(base, RPW)], tvec_v)
    pltpu.sync_copy(ctab.at[0, pl.ds(l0, NL)], cbig.at[pl.ds(0, NL)])
    pltpu.sync_copy(ctab.at[1, pl.ds(l0, NL)], cbig.at[pl.ds(NL, NL)])
    pltpu.sync_copy(gamma, gb_v.at[0])
    pltpu.sync_copy(beta, gb_v.at[1])

    gvecs = [gb_v[0, pl.ds(j * LANES, LANES)] for j in range(DJ)]
    bvecs = [gb_v[1, pl.ds(j * LANES, LANES)] for j in range(DJ)]
    iota = lax.iota(jnp.int32, LANES)
    m15 = iota == (LANES - 1)
    kfulls = [jnp.full((LANES,), k, jnp.int32) for k in range(LANES)]

    def issue_gather(chunk, xb, sem):
        for j in range(NSUB):
            pltpu.async_copy(
                word.at[idx_v.at[pl.ds(chunk * CHUNK + j * SUB, SUB)]],
                xb.at[pl.ds(j * SUB, SUB)], sem)

    def drain_gather(xb, sem):
        pltpu.make_async_copy(word.at[pl.ds(0, CHUNK)], xb, sem).wait()

    def splat(vec, kfull):
        return vec.at[kfull].get(mode="promise_in_bounds")

    def compute(chunk, xb):
        li = (base + chunk * CHUNK) // B - l0
        c0s = [cbig[li, pl.ds(j * LANES, LANES)] for j in range(DJ)]
        c1s = [cbig[NL + li, pl.ds(j * LANES, LANES)] for j in range(DJ)]

        @pl.loop(0, GROUPS)
        def _group(g):
            r0 = g * LANES
            tvec = tvec_v[pl.ds(chunk * CHUNK + r0, LANES)]
            for k in range(LANES):
                r = r0 + k
                tmask = splat(tvec, kfulls[k]) != 0
                xs = [xb[r, pl.ds(j * LANES, LANES)]
                      + jnp.where(tmask, c1s[j], c0s[j])
                      for j in range(DJ)]
                tot = (xs[0] + xs[1]) + (xs[2] + xs[3])
                sq = [x * x for x in xs]
                tot2 = (sq[0] + sq[1]) + (sq[2] + sq[3])
                plsc.store_scatter(srow_v, [kfulls[k]], plsc.cumsum(tot),
                                   mask=m15)
                plsc.store_scatter(sqrow_v, [kfulls[k]], plsc.cumsum(tot2),
                                   mask=m15)
                for j in range(DJ):
                    xb[r, pl.ds(j * LANES, LANES)] = xs[j]
            mean16 = srow_v[...] * (1.0 / D)
            ex216 = sqrow_v[...] * (1.0 / D)
            var = jnp.maximum(ex216 - mean16 * mean16, 0.0) + EPS
            yi = (jnp.int32(0x5F3759DF)
                  - (plsc.bitcast(var, jnp.int32) >> 1))
            y = plsc.bitcast(yi, jnp.float32)
            for _ in range(3):
                y = y * (1.5 - 0.5 * var * y * y)
            for k in range(LANES):
                r = r0 + k
                meank = splat(mean16, kfulls[k])
                invk = splat(y, kfulls[k])
                for j in range(DJ):
                    xj = xb[r, pl.ds(j * LANES, LANES)]
                    xb[r, pl.ds(j * LANES, LANES)] = (
                        (xj - meank) * invk * gvecs[j] + bvecs[j])

    def out_slice(chunk):
        gbase = base + chunk * CHUNK
        return out.at[gbase // B, pl.ds(gbase % B, CHUNK)]

    def drain_out(xb, sem):
        pltpu.make_async_copy(out_slice(0), xb, sem).wait()

    NB = 4
    issue_gather(0, xbufs[0], gsems[0])
    for b in range(1, NB):
        pltpu.async_copy(xbufs[b], out_slice(b), osems[b])

    @pl.loop(0, NCH, step=NB)
    def _chunks(ci):
        for b in range(NB):
            chunk = ci + b
            xb, sem = xbufs[b], gsems[b]
            nb = (b + 1) % NB

            def prefetch():
                drain_out(xbufs[nb], osems[nb])
                issue_gather(chunk + 1, xbufs[nb], gsems[nb])

            if b < NB - 1:
                prefetch()
            else:
                @pl.when(chunk + 1 < NCH)
                def _():
                    prefetch()
            drain_gather(xb, sem)
            compute(chunk, xb)
            pltpu.async_copy(xb, out_slice(chunk), osems[b])

    for b in range(NB):
        drain_out(xbufs[b], osems[b])


@jax.jit
def _emb(ids, tts, ctab, word, gamma, beta):
    mesh = plsc.VectorSubcoreMesh(core_axis_name="c", subcore_axis_name="s",
                                  num_cores=NC, num_subcores=NS)
    return pl.kernel(
        _emb_body,
        out_type=jax.ShapeDtypeStruct((L, B, D), jnp.float32),
        mesh=mesh,
        compiler_params=pltpu.CompilerParams(needs_layout_passes=False,
                                             use_tc_tiling_on_sc=False),
        scratch_types=[
            pltpu.VMEM((RPW,), jnp.int32),
            pltpu.VMEM((RPW,), jnp.int32),
            pltpu.VMEM((2 * NL, D), jnp.float32),
            pltpu.VMEM((2, D), jnp.float32),
            [pltpu.VMEM((CHUNK, D), jnp.float32)
             for _ in range(4)],
            pltpu.VMEM((LANES,), jnp.float32),
            pltpu.VMEM((LANES,), jnp.float32),
            [pltpu.SemaphoreType.DMA for _ in range(4)],
            [pltpu.SemaphoreType.DMA for _ in range(4)],
        ],
    )(ids, tts, ctab, word, gamma, beta)


def kernel(input_ids, token_type_ids, word_table, pos_table, type_table,
           gamma, beta):
    ids = input_ids.astype(jnp.int32).T.reshape(N)
    tts = token_type_ids.astype(jnp.int32).T.reshape(N)
    ctab = jnp.zeros((T, L + NL, D), jnp.float32)
    ctab = ctab.at[:, :L, :].set(type_table[:, None, :]
                                 + pos_table[None, :L, :])
    q = _emb(ids, tts, ctab, word_table, gamma, beta)
    return q.transpose(1, 0, 2)
